# baseline (speedup 1.0000x reference)
import functools

import jax
import jax.numpy as jnp
from jax import lax
from jax.experimental import pallas as pl
from jax.experimental.pallas import tpu as pltpu
from jax.experimental.pallas import tpu_sc as plsc

NUM_SEG = 10000
PAD_SEG = 10240
D = 128
ROWS = 320000
G = 80
NBUF = 4
NGROUPS = ROWS // G
NWORK = 32
GMAIN = NGROUPS // NWORK
SEG_PER_SUB = PAD_SEG // 16
CW = 4


def _sc_body(feat_hbm, seg_hbm, zs_hbm, zc_hbm, ones_hbm,
             out_s, out_c, acc, cnt, fbuf, ibuf, ones_v, zbuf,
             fsem, isem, csem, wsem):
    c = lax.axis_index("c")
    s = lax.axis_index("s")
    wid = c * 16 + s
    base = s * SEG_PER_SUB

    pltpu.sync_copy(zs_hbm, fbuf.at[0])
    pltpu.sync_copy(zc_hbm, zbuf)
    pltpu.sync_copy(ones_hbm, ones_v)
    for j in range(SEG_PER_SUB // G):
        pltpu.async_copy(fbuf.at[0], acc.at[pl.ds(base + j * G, G)], wsem)
        pltpu.async_copy(zbuf, cnt.at[pl.ds(base + j * G, G)], wsem)
    for j in range(SEG_PER_SUB // G):
        pltpu.make_async_copy(fbuf.at[0], acc.at[pl.ds(base + j * G, G)],
                              wsem).wait()
        pltpu.make_async_copy(zbuf, cnt.at[pl.ds(base + j * G, G)],
                              wsem).wait()

    g0 = wid * GMAIN

    def start_read(gabs, b):
        r0 = gabs * G
        pltpu.async_copy(feat_hbm.at[pl.ds(r0, G)], fbuf.at[b], fsem.at[b])
        pltpu.async_copy(seg_hbm.at[pl.ds(r0, G)], ibuf.at[b], isem.at[b])

    def wait_read(b):
        pltpu.make_async_copy(feat_hbm.at[pl.ds(0, G)], fbuf.at[b],
                              fsem.at[b]).wait()
        pltpu.make_async_copy(seg_hbm.at[pl.ds(0, G)], ibuf.at[b],
                              isem.at[b]).wait()

    def scatter(b):
        cdesc = pltpu.async_copy(ones_v, cnt.at[ibuf.at[b]], csem.at[b],
                                 add=True)
        pltpu.sync_copy(fbuf.at[b], acc.at[ibuf.at[b]], add=True)
        cdesc.wait()

    for b in range(NBUF):
        start_read(g0 + b, b)
    plsc.subcore_barrier()

    def body(i, carry):
        for b in range(NBUF):
            g = NBUF * i + b
            wait_read(b)
            scatter(b)
            start_read(jnp.minimum(g0 + g + NBUF, NGROUPS - 1), b)
        return carry

    nfull = GMAIN // NBUF
    lax.fori_loop(0, nfull, body, 0)
    for k in range(GMAIN - nfull * NBUF):
        slot = nfull * NBUF + k
        b = slot % NBUF
        wait_read(b)
        scatter(b)
    for slot in range(GMAIN, nfull * NBUF + NBUF):
        wait_read(slot % NBUF)

    plsc.subcore_barrier()
    for j in range(SEG_PER_SUB // G):
        pltpu.async_copy(acc.at[pl.ds(base + j * G, G)],
                         out_s.at[c, pl.ds(base + j * G, G)], wsem)
        pltpu.async_copy(cnt.at[pl.ds(base + j * G, G)],
                         out_c.at[c, pl.ds(base + j * G, G)], wsem)
    for j in range(SEG_PER_SUB // G):
        pltpu.make_async_copy(acc.at[pl.ds(base + j * G, G)],
                              out_s.at[c, pl.ds(base + j * G, G)],
                              wsem).wait()
        pltpu.make_async_copy(cnt.at[pl.ds(base + j * G, G)],
                              out_c.at[c, pl.ds(base + j * G, G)],
                              wsem).wait()


_sc_partial = functools.partial(
    pl.kernel,
    out_type=(jax.ShapeDtypeStruct((2, PAD_SEG, D), jnp.float32),
              jax.ShapeDtypeStruct((2, PAD_SEG, CW), jnp.float32)),
    mesh=plsc.VectorSubcoreMesh(core_axis_name="c", subcore_axis_name="s"),
    compiler_params=pltpu.CompilerParams(use_tc_tiling_on_sc=False),
    scratch_types=[
        pltpu.VMEM_SHARED((PAD_SEG, D), jnp.float32),
        pltpu.VMEM_SHARED((PAD_SEG, CW), jnp.float32),
        pltpu.VMEM((NBUF, G, D), jnp.float32),
        pltpu.VMEM((NBUF, G), jnp.int32),
        pltpu.VMEM((G, CW), jnp.float32),
        pltpu.VMEM((G, CW), jnp.float32),
        pltpu.SemaphoreType.DMA((NBUF,)),
        pltpu.SemaphoreType.DMA((NBUF,)),
        pltpu.SemaphoreType.DMA((NBUF,)),
        pltpu.SemaphoreType.DMA,
    ],
)(_sc_body)


def _merge_body(s_ref, c_ref, o_ref):
    tot = s_ref[0] + s_ref[1]
    n = c_ref[0, :, 0:1] + c_ref[1, :, 0:1]
    o_ref[...] = tot / jnp.maximum(n, 1.0)


def _merge(sums, cnts):
    blk = 1000
    return pl.pallas_call(
        _merge_body,
        grid=(NUM_SEG // blk,),
        in_specs=[
            pl.BlockSpec((2, blk, D), lambda i: (0, i, 0)),
            pl.BlockSpec((2, blk, CW), lambda i: (0, i, 0)),
        ],
        out_specs=pl.BlockSpec((blk, D), lambda i: (i, 0)),
        out_shape=jax.ShapeDtypeStruct((NUM_SEG, D), jnp.float32),
    )(sums, cnts)


def kernel(features, segments):
    zs = jnp.zeros((G, D), jnp.float32)
    zc = jnp.zeros((G, CW), jnp.float32)
    ones = jnp.zeros((G, CW), jnp.float32).at[:, 0].set(1.0)
    sums, cnts = _sc_partial(features, segments.astype(jnp.int32),
                             zs, zc, ones)
    return _merge(sums, cnts)


# speedup vs baseline: 1.0102x; 1.0102x over previous
import functools

import jax
import jax.numpy as jnp
from jax import lax
from jax.experimental import pallas as pl
from jax.experimental.pallas import tpu as pltpu
from jax.experimental.pallas import tpu_sc as plsc

NUM_SEG = 10000
PAD_SEG = 10240
D = 128
ROWS = 320000
G = 64
NBUF = 4
NGROUPS = ROWS // G
NWORK = 32
GMAIN = NGROUPS // NWORK
NEXTRA = NGROUPS - GMAIN * NWORK
SEG_PER_SUB = PAD_SEG // 16
CW = 8


def _sc_body(feat_hbm, seg_hbm, zs_hbm, zc_hbm, ones_hbm,
             out_s, out_c, acc, cnt, fbuf, ibuf, ones_v, zbuf,
             fsem, isem, csem, wsem):
    c = lax.axis_index("c")
    s = lax.axis_index("s")
    wid = c * 16 + s
    base = s * SEG_PER_SUB

    pltpu.sync_copy(zs_hbm, fbuf.at[0])
    pltpu.sync_copy(zc_hbm, zbuf)
    pltpu.sync_copy(ones_hbm, ones_v)
    for j in range(SEG_PER_SUB // G):
        pltpu.sync_copy(fbuf.at[0], acc.at[pl.ds(base + j * G, G)])
        pltpu.sync_copy(zbuf, cnt.at[pl.ds(base + j * G, G)])

    g0 = wid * GMAIN

    def start_read(gabs, b):
        r0 = gabs * G
        pltpu.async_copy(feat_hbm.at[pl.ds(r0, G)], fbuf.at[b], fsem.at[b])
        pltpu.async_copy(seg_hbm.at[pl.ds(r0, G)], ibuf.at[b], isem.at[b])

    def wait_read(b):
        pltpu.make_async_copy(feat_hbm.at[pl.ds(0, G)], fbuf.at[b],
                              fsem.at[b]).wait()
        pltpu.make_async_copy(seg_hbm.at[pl.ds(0, G)], ibuf.at[b],
                              isem.at[b]).wait()

    def scatter(b):
        cdesc = pltpu.async_copy(ones_v, cnt.at[ibuf.at[b]], csem.at[b],
                                 add=True)
        pltpu.sync_copy(fbuf.at[b], acc.at[ibuf.at[b]], add=True)
        cdesc.wait()

    for b in range(NBUF):
        start_read(g0 + b, b)
    plsc.subcore_barrier()

    def body(i, carry):
        for b in range(NBUF):
            g = NBUF * i + b
            wait_read(b)
            scatter(b)
            start_read(g0 + g + NBUF, b)
        return carry

    lax.fori_loop(0, GMAIN // NBUF, body, 0)
    for b in range(NBUF):
        wait_read(b)

    @pl.when(wid >= NWORK - NEXTRA)
    def _():
        gabs = GMAIN * NWORK + (wid - (NWORK - NEXTRA))
        start_read(gabs, 0)
        wait_read(0)
        scatter(0)

    plsc.subcore_barrier()
    for j in range(SEG_PER_SUB // G):
        pltpu.async_copy(acc.at[pl.ds(base + j * G, G)],
                         out_s.at[c, pl.ds(base + j * G, G)], wsem)
        pltpu.async_copy(cnt.at[pl.ds(base + j * G, G)],
                         out_c.at[c, pl.ds(base + j * G, G)], wsem)
    for j in range(SEG_PER_SUB // G):
        pltpu.make_async_copy(acc.at[pl.ds(base + j * G, G)],
                              out_s.at[c, pl.ds(base + j * G, G)],
                              wsem).wait()
        pltpu.make_async_copy(cnt.at[pl.ds(base + j * G, G)],
                              out_c.at[c, pl.ds(base + j * G, G)],
                              wsem).wait()


_sc_partial = functools.partial(
    pl.kernel,
    out_type=(jax.ShapeDtypeStruct((2, PAD_SEG, D), jnp.float32),
              jax.ShapeDtypeStruct((2, PAD_SEG, CW), jnp.float32)),
    mesh=plsc.VectorSubcoreMesh(core_axis_name="c", subcore_axis_name="s"),
    compiler_params=pltpu.CompilerParams(use_tc_tiling_on_sc=False),
    scratch_types=[
        pltpu.VMEM_SHARED((PAD_SEG, D), jnp.float32),
        pltpu.VMEM_SHARED((PAD_SEG, CW), jnp.float32),
        pltpu.VMEM((NBUF, G, D), jnp.float32),
        pltpu.VMEM((NBUF, G), jnp.int32),
        pltpu.VMEM((G, CW), jnp.float32),
        pltpu.VMEM((G, CW), jnp.float32),
        pltpu.SemaphoreType.DMA((NBUF,)),
        pltpu.SemaphoreType.DMA((NBUF,)),
        pltpu.SemaphoreType.DMA((NBUF,)),
        pltpu.SemaphoreType.DMA,
    ],
)(_sc_body)


def _merge_body(s_ref, c_ref, o_ref):
    tot = s_ref[0] + s_ref[1]
    n = c_ref[0, :, 0:1] + c_ref[1, :, 0:1]
    o_ref[...] = tot / jnp.maximum(n, 1.0)


def _merge(sums, cnts):
    blk = 1000
    return pl.pallas_call(
        _merge_body,
        grid=(NUM_SEG // blk,),
        in_specs=[
            pl.BlockSpec((2, blk, D), lambda i: (0, i, 0)),
            pl.BlockSpec((2, blk, CW), lambda i: (0, i, 0)),
        ],
        out_specs=pl.BlockSpec((blk, D), lambda i: (i, 0)),
        out_shape=jax.ShapeDtypeStruct((NUM_SEG, D), jnp.float32),
    )(sums, cnts)


def kernel(features, segments):
    zs = jnp.zeros((G, D), jnp.float32)
    zc = jnp.zeros((G, CW), jnp.float32)
    ones = jnp.zeros((G, CW), jnp.float32).at[:, 0].set(1.0)
    sums, cnts = _sc_partial(features, segments.astype(jnp.int32),
                             zs, zc, ones)
    return _merge(sums, cnts)


# speedup vs baseline: 1.0150x; 1.0047x over previous
import functools

import jax
import jax.numpy as jnp
from jax import lax
from jax.experimental import pallas as pl
from jax.experimental.pallas import tpu as pltpu
from jax.experimental.pallas import tpu_sc as plsc

NUM_SEG = 10000
PAD_SEG = 10240
D = 128
ROWS = 320000
G = 64
NBUF = 4
NGROUPS = ROWS // G
NWORK = 32
GMAIN = NGROUPS // NWORK
NEXTRA = NGROUPS - GMAIN * NWORK
SEG_PER_SUB = PAD_SEG // 16
CW = 8


def _sc_body(feat_hbm, seg_hbm, zs_hbm, zc_hbm, ones_hbm,
             out_s, out_c, acc, cnt, fbuf, ibuf, ones_v, zbuf,
             fsem, isem, csem, wsem):
    c = lax.axis_index("c")
    s = lax.axis_index("s")
    wid = c * 16 + s
    base = s * SEG_PER_SUB

    pltpu.sync_copy(zs_hbm, fbuf.at[0])
    pltpu.sync_copy(zc_hbm, zbuf)
    pltpu.sync_copy(ones_hbm, ones_v)
    for j in range(SEG_PER_SUB // G):
        pltpu.async_copy(fbuf.at[0], acc.at[pl.ds(base + j * G, G)], wsem)
        pltpu.async_copy(zbuf, cnt.at[pl.ds(base + j * G, G)], wsem)
    for j in range(SEG_PER_SUB // G):
        pltpu.make_async_copy(fbuf.at[0], acc.at[pl.ds(base + j * G, G)],
                              wsem).wait()
        pltpu.make_async_copy(zbuf, cnt.at[pl.ds(base + j * G, G)],
                              wsem).wait()

    g0 = wid * GMAIN

    def start_read(gabs, b):
        r0 = gabs * G
        pltpu.async_copy(feat_hbm.at[pl.ds(r0, G)], fbuf.at[b], fsem.at[b])
        pltpu.async_copy(seg_hbm.at[pl.ds(r0, G)], ibuf.at[b], isem.at[b])

    def wait_read(b):
        pltpu.make_async_copy(feat_hbm.at[pl.ds(0, G)], fbuf.at[b],
                              fsem.at[b]).wait()
        pltpu.make_async_copy(seg_hbm.at[pl.ds(0, G)], ibuf.at[b],
                              isem.at[b]).wait()

    def scatter(b):
        cdesc = pltpu.async_copy(ones_v, cnt.at[ibuf.at[b]], csem.at[b],
                                 add=True)
        pltpu.sync_copy(fbuf.at[b], acc.at[ibuf.at[b]], add=True)
        cdesc.wait()

    for b in range(NBUF):
        start_read(g0 + b, b)
    plsc.subcore_barrier()

    def body(i, carry):
        for b in range(NBUF):
            g = NBUF * i + b
            wait_read(b)
            scatter(b)
            start_read(g0 + g + NBUF, b)
        return carry

    lax.fori_loop(0, GMAIN // NBUF, body, 0)
    for b in range(NBUF):
        wait_read(b)

    @pl.when(wid >= NWORK - NEXTRA)
    def _():
        gabs = GMAIN * NWORK + (wid - (NWORK - NEXTRA))
        start_read(gabs, 0)
        wait_read(0)
        scatter(0)

    plsc.subcore_barrier()
    for j in range(SEG_PER_SUB // G):
        pltpu.async_copy(acc.at[pl.ds(base + j * G, G)],
                         out_s.at[c, pl.ds(base + j * G, G)], wsem)
        pltpu.async_copy(cnt.at[pl.ds(base + j * G, G)],
                         out_c.at[c, pl.ds(base + j * G, G)], wsem)
    for j in range(SEG_PER_SUB // G):
        pltpu.make_async_copy(acc.at[pl.ds(base + j * G, G)],
                              out_s.at[c, pl.ds(base + j * G, G)],
                              wsem).wait()
        pltpu.make_async_copy(cnt.at[pl.ds(base + j * G, G)],
                              out_c.at[c, pl.ds(base + j * G, G)],
                              wsem).wait()


_sc_partial = functools.partial(
    pl.kernel,
    out_type=(jax.ShapeDtypeStruct((2, PAD_SEG, D), jnp.float32),
              jax.ShapeDtypeStruct((2, PAD_SEG, CW), jnp.float32)),
    mesh=plsc.VectorSubcoreMesh(core_axis_name="c", subcore_axis_name="s"),
    compiler_params=pltpu.CompilerParams(use_tc_tiling_on_sc=False),
    scratch_types=[
        pltpu.VMEM_SHARED((PAD_SEG, D), jnp.float32),
        pltpu.VMEM_SHARED((PAD_SEG, CW), jnp.float32),
        pltpu.VMEM((NBUF, G, D), jnp.float32),
        pltpu.VMEM((NBUF, G), jnp.int32),
        pltpu.VMEM((G, CW), jnp.float32),
        pltpu.VMEM((G, CW), jnp.float32),
        pltpu.SemaphoreType.DMA((NBUF,)),
        pltpu.SemaphoreType.DMA((NBUF,)),
        pltpu.SemaphoreType.DMA((NBUF,)),
        pltpu.SemaphoreType.DMA,
    ],
)(_sc_body)


def _merge_body(s_ref, c_ref, o_ref):
    tot = s_ref[0] + s_ref[1]
    n = c_ref[0, :, 0:1] + c_ref[1, :, 0:1]
    o_ref[...] = tot / jnp.maximum(n, 1.0)


def _merge(sums, cnts):
    blk = 1000
    return pl.pallas_call(
        _merge_body,
        grid=(NUM_SEG // blk,),
        in_specs=[
            pl.BlockSpec((2, blk, D), lambda i: (0, i, 0)),
            pl.BlockSpec((2, blk, CW), lambda i: (0, i, 0)),
        ],
        out_specs=pl.BlockSpec((blk, D), lambda i: (i, 0)),
        out_shape=jax.ShapeDtypeStruct((NUM_SEG, D), jnp.float32),
    )(sums, cnts)


def kernel(features, segments):
    zs = jnp.zeros((G, D), jnp.float32)
    zc = jnp.zeros((G, CW), jnp.float32)
    ones = jnp.zeros((G, CW), jnp.float32).at[:, 0].set(1.0)
    sums, cnts = _sc_partial(features, segments.astype(jnp.int32),
                             zs, zc, ones)
    return _merge(sums, cnts)
